# baseline (speedup 1.0000x reference)
import functools

import jax
import jax.numpy as jnp
from jax import lax
from jax.experimental import pallas as pl
from jax.experimental.pallas import tpu as pltpu
from jax.experimental.pallas import tpu_sc as plsc

B, H, W = 64, 80, 80
P = B * H * W
M = 4
CH = 20
NW = 32
HROWS = 16
SLAB = HROWS * W
NSLAB = (P // NW) // SLAB
SGROUPS = SLAB // 16

LN2 = 0.6931471805599453
LOG2PI = 1.8378770664093453

_LOGC = (-0.4842381066803376, 0.2456068793580864, -0.0683422317501495)


def _flog(y):
    yi = lax.bitcast_convert_type(y, jnp.int32)
    e = lax.shift_right_arithmetic(yi, 23) - 127
    mi = jnp.bitwise_or(jnp.bitwise_and(yi, 0x007FFFFF), 0x3F800000)
    f = lax.bitcast_convert_type(mi, jnp.float32) - 1.0
    p = jnp.float32(_LOGC[2])
    p = p * f + _LOGC[1]
    p = p * f + _LOGC[0]
    return e.astype(jnp.float32) * LN2 + (p * (f * f) + f)


def _rcp(x):
    xi = lax.bitcast_convert_type(x, jnp.int32)
    r = lax.bitcast_convert_type(0x7EF311C3 - xi, jnp.float32)
    r = r * (2.0 - x * r)
    r = r * (2.0 - x * r)
    return r


def _softplus(x):
    t = jnp.exp(-jnp.abs(x))
    p = jnp.float32(_LOGC[2])
    p = p * t + _LOGC[1]
    p = p * t + _LOGC[0]
    return jnp.maximum(x, 0.0) + (p * (t * t) + t)


def _body(x_hbm, tl_hbm, ns_hbm, out_hbm, x_v, tl_v, ns_v, out_v,
          xsem, tsem, nsem):
    cid = lax.axis_index("c")
    sid = lax.axis_index("s")
    wid = sid * 2 + cid

    def _addr(j):
        b = 2 * wid + j // 5
        h0 = (j % 5) * HROWS
        return b, h0

    def _issue(j, p):
        b, h0 = _addr(j)
        pltpu.async_copy(x_hbm.at[b, :, pl.ds(h0, HROWS), :], x_v.at[p], xsem)
        pltpu.async_copy(tl_hbm.at[b, pl.ds(h0, HROWS)], tl_v.at[p], tsem)
        pltpu.async_copy(ns_hbm.at[b, pl.ds(h0, HROWS)], ns_v.at[p], nsem)

    _issue(0, 0)

    def slab_body(j, _):
        p = j % 2
        b, h0 = _addr(j)
        pltpu.make_async_copy(
            x_hbm.at[b, :, pl.ds(h0, HROWS), :], x_v.at[p], xsem).wait()
        pltpu.make_async_copy(
            tl_hbm.at[b, pl.ds(h0, HROWS)], tl_v.at[p], tsem).wait()
        pltpu.make_async_copy(
            ns_hbm.at[b, pl.ds(h0, HROWS)], ns_v.at[p], nsem).wait()

        @pl.when(j < NSLAB - 1)
        def _prefetch():
            _issue(j + 1, 1 - p)

        def px_group(i, _):
            r = i // 5
            c0 = (i % 5) * 16
            s = pl.ds(c0, 16)
            n = ns_v[p, r, s]
            on_b = n > 0
            valid = [t < n for t in range(M)]

            tx = [tl_v[p, r, 2 * t, s] for t in range(M)]
            ty = [tl_v[p, r, 2 * t + 1, s] for t in range(M)]

            s_spl = jnp.zeros((16,), jnp.float32)
            s_l = jnp.zeros((16,), jnp.float32)
            s_nll = jnp.zeros((16,), jnp.float32)
            for k in range(M):
                l = x_v[p, 5 * k, r, s]
                mux = x_v[p, 5 * k + 1, r, s]
                muy = x_v[p, 5 * k + 2, r, s]
                sdxr = x_v[p, 5 * k + 3, r, s]
                sdyr = x_v[p, 5 * k + 4, r, s]

                ex = 1.0 / (1.0 + jnp.exp(-mux))
                ey = 1.0 / (1.0 + jnp.exp(-muy))
                sdx = _softplus(sdxr) + 1e-4
                sdy = _softplus(sdyr) + 1e-4

                dx = ex - tx[0]
                dy = ey - ty[0]
                best = jnp.where(valid[0], dx * dx + dy * dy, 1e9)
                bx = tx[0]
                by = ty[0]
                for t in range(1, M):
                    dx = ex - tx[t]
                    dy = ey - ty[t]
                    d2 = jnp.where(valid[t], dx * dx + dy * dy, 1e9)
                    mlt = d2 < best
                    best = jnp.where(mlt, d2, best)
                    bx = jnp.where(mlt, tx[t], bx)
                    by = jnp.where(mlt, ty[t], by)

                zx = (bx - ex) / sdx
                zy = (by - ey) / sdy
                s_spl = s_spl + _softplus(l)
                s_l = s_l + l
                s_nll = s_nll + (0.5 * (zx * zx + zy * zy)
                                 + _flog(sdx * sdy) + LOG2PI)

            out_v[r, s] = jnp.where(on_b, s_spl - s_l + s_nll, 0.1 * s_spl)
            return 0

        lax.fori_loop(0, SGROUPS, px_group, 0, unroll=2)
        pltpu.sync_copy(out_v, out_hbm.at[b, pl.ds(h0, HROWS)])
        return 0

    lax.fori_loop(0, NSLAB, slab_body, 0, unroll=False)


@jax.jit
def _sc_nll(x2d, tl2d, ns3d):
    mesh = plsc.VectorSubcoreMesh(core_axis_name="c", subcore_axis_name="s")
    f = functools.partial(
        pl.kernel,
        mesh=mesh,
        compiler_params=pltpu.CompilerParams(needs_layout_passes=False),
        out_type=jax.ShapeDtypeStruct((B, H, W), jnp.float32),
        scratch_types=[
            pltpu.VMEM((2, CH, HROWS, W), jnp.float32),
            pltpu.VMEM((2, HROWS, 2 * M, W), jnp.float32),
            pltpu.VMEM((2, HROWS, W), jnp.int32),
            pltpu.VMEM((HROWS, W), jnp.float32),
            pltpu.SemaphoreType.DMA,
            pltpu.SemaphoreType.DMA,
            pltpu.SemaphoreType.DMA,
        ],
    )(_body)
    return f(x2d, tl2d, ns3d)


def kernel(x_cat, true_locs, true_n_sources, topk):
    x4 = jnp.transpose(x_cat, (0, 3, 1, 2))
    tl4 = jnp.transpose(true_locs, (0, 1, 3, 4, 2)).reshape(B, H, 2 * M, W)
    ns3d = true_n_sources.astype(jnp.int32)
    return _sc_nll(x4, tl4, ns3d)


# speedup vs baseline: 1.3306x; 1.3306x over previous
import functools

import jax
import jax.numpy as jnp
from jax import lax
from jax.experimental import pallas as pl
from jax.experimental.pallas import tpu as pltpu
from jax.experimental.pallas import tpu_sc as plsc

B, H, W = 64, 80, 80
P = B * H * W
M = 4
CH = 20
NW = 32
HROWS = 16
SLAB = HROWS * W
NSLAB = (P // NW) // SLAB
SGROUPS = SLAB // 16

LN2 = 0.6931471805599453
LOG2PI = 1.8378770664093453

_LOGC = (-0.4842381066803376, 0.2456068793580864, -0.0683422317501495)


def _flog(y):
    yi = lax.bitcast_convert_type(y, jnp.int32)
    e = lax.shift_right_arithmetic(yi, 23) - 127
    mi = jnp.bitwise_or(jnp.bitwise_and(yi, 0x007FFFFF), 0x3F800000)
    f = lax.bitcast_convert_type(mi, jnp.float32) - 1.0
    p = jnp.float32(_LOGC[2])
    p = p * f + _LOGC[1]
    p = p * f + _LOGC[0]
    return e.astype(jnp.float32) * LN2 + (p * (f * f) + f)


def _rcp(x):
    xi = lax.bitcast_convert_type(x, jnp.int32)
    r = lax.bitcast_convert_type(0x7EF311C3 - xi, jnp.float32)
    r = r * (2.0 - x * r)
    r = r * (2.0 - x * r)
    return r


def _softplus(x):
    t = jnp.exp(-jnp.abs(x))
    p = jnp.float32(_LOGC[2])
    p = p * t + _LOGC[1]
    p = p * t + _LOGC[0]
    return jnp.maximum(x, 0.0) + (p * (t * t) + t)


def _body(x_hbm, tl_hbm, ns_hbm, out_hbm, x_v, tl_v, ns_v, out_v,
          xsem, tsem, nsem):
    cid = lax.axis_index("c")
    sid = lax.axis_index("s")
    wid = sid * 2 + cid

    def _addr(j):
        b = 2 * wid + j // 5
        h0 = (j % 5) * HROWS
        return b, h0

    def _issue(j, p):
        b, h0 = _addr(j)
        pltpu.async_copy(x_hbm.at[b, :, pl.ds(h0, HROWS), :], x_v.at[p], xsem)
        pltpu.async_copy(tl_hbm.at[b, pl.ds(h0, HROWS)], tl_v.at[p], tsem)
        pltpu.async_copy(ns_hbm.at[b, pl.ds(h0, HROWS)], ns_v.at[p], nsem)

    _issue(0, 0)

    def slab_body(j, _):
        p = j % 2
        b, h0 = _addr(j)
        pltpu.make_async_copy(
            x_hbm.at[b, :, pl.ds(h0, HROWS), :], x_v.at[p], xsem).wait()
        pltpu.make_async_copy(
            tl_hbm.at[b, pl.ds(h0, HROWS)], tl_v.at[p], tsem).wait()
        pltpu.make_async_copy(
            ns_hbm.at[b, pl.ds(h0, HROWS)], ns_v.at[p], nsem).wait()

        @pl.when(j < NSLAB - 1)
        def _prefetch():
            _issue(j + 1, 1 - p)

        def px_group(i, _):
            r = i // 5
            c0 = (i % 5) * 16
            s = pl.ds(c0, 16)
            n = ns_v[p, r, s]
            on_b = n > 0
            valid = [t < n for t in range(M)]

            tx = [tl_v[p, r, 2 * t, s] for t in range(M)]
            ty = [tl_v[p, r, 2 * t + 1, s] for t in range(M)]

            s_spl = jnp.zeros((16,), jnp.float32)
            s_l = jnp.zeros((16,), jnp.float32)
            s_nll = jnp.zeros((16,), jnp.float32)
            for k in range(M):
                l = x_v[p, 5 * k, r, s]
                mux = x_v[p, 5 * k + 1, r, s]
                muy = x_v[p, 5 * k + 2, r, s]
                sdxr = x_v[p, 5 * k + 3, r, s]
                sdyr = x_v[p, 5 * k + 4, r, s]

                ex = 1.0 / (1.0 + jnp.exp(-mux))
                ey = 1.0 / (1.0 + jnp.exp(-muy))
                sdx = _softplus(sdxr) + 1e-4
                sdy = _softplus(sdyr) + 1e-4

                dx = ex - tx[0]
                dy = ey - ty[0]
                best = jnp.where(valid[0], dx * dx + dy * dy, 1e9)
                bx = tx[0]
                by = ty[0]
                for t in range(1, M):
                    dx = ex - tx[t]
                    dy = ey - ty[t]
                    d2 = jnp.where(valid[t], dx * dx + dy * dy, 1e9)
                    mlt = d2 < best
                    best = jnp.where(mlt, d2, best)
                    bx = jnp.where(mlt, tx[t], bx)
                    by = jnp.where(mlt, ty[t], by)

                zx = (bx - ex) / sdx
                zy = (by - ey) / sdy
                s_spl = s_spl + _softplus(l)
                s_l = s_l + l
                s_nll = s_nll + (0.5 * (zx * zx + zy * zy)
                                 + _flog(sdx * sdy) + LOG2PI)

            out_v[r, s] = jnp.where(on_b, s_spl - s_l + s_nll, 0.1 * s_spl)
            return 0

        lax.fori_loop(0, SGROUPS, px_group, 0, unroll=False)
        pltpu.sync_copy(out_v, out_hbm.at[b, pl.ds(h0, HROWS)])
        return 0

    lax.fori_loop(0, NSLAB, slab_body, 0, unroll=False)


@jax.jit
def _sc_nll(x2d, tl2d, ns3d):
    mesh = plsc.VectorSubcoreMesh(core_axis_name="c", subcore_axis_name="s")
    f = functools.partial(
        pl.kernel,
        mesh=mesh,
        compiler_params=pltpu.CompilerParams(needs_layout_passes=False),
        out_type=jax.ShapeDtypeStruct((B, H, W), jnp.float32),
        scratch_types=[
            pltpu.VMEM((2, CH, HROWS, W), jnp.float32),
            pltpu.VMEM((2, HROWS, 2 * M, W), jnp.float32),
            pltpu.VMEM((2, HROWS, W), jnp.int32),
            pltpu.VMEM((HROWS, W), jnp.float32),
            pltpu.SemaphoreType.DMA,
            pltpu.SemaphoreType.DMA,
            pltpu.SemaphoreType.DMA,
        ],
    )(_body)
    return f(x2d, tl2d, ns3d)


def kernel(x_cat, true_locs, true_n_sources, topk):
    x4 = jnp.transpose(x_cat, (0, 3, 1, 2))
    tl4 = jnp.transpose(true_locs, (0, 1, 3, 4, 2)).reshape(B, H, 2 * M, W)
    ns3d = true_n_sources.astype(jnp.int32)
    return _sc_nll(x4, tl4, ns3d)


# speedup vs baseline: 1.3323x; 1.0013x over previous
import functools

import jax
import jax.numpy as jnp
from jax import lax
from jax.experimental import pallas as pl
from jax.experimental.pallas import tpu as pltpu
from jax.experimental.pallas import tpu_sc as plsc

B, H, W = 64, 80, 80
P = B * H * W
M = 4
CH = 20
NW = 32
HROWS = 16
SLAB = HROWS * W
NSLAB = (P // NW) // SLAB
SGROUPS = SLAB // 16

LN2 = 0.6931471805599453
LOG2PI = 1.8378770664093453

_LOGC = (-0.4842381066803376, 0.2456068793580864, -0.0683422317501495)


def _flog(y):
    yi = lax.bitcast_convert_type(y, jnp.int32)
    e = lax.shift_right_arithmetic(yi, 23) - 127
    mi = jnp.bitwise_or(jnp.bitwise_and(yi, 0x007FFFFF), 0x3F800000)
    f = lax.bitcast_convert_type(mi, jnp.float32) - 1.0
    p = jnp.float32(_LOGC[2])
    p = p * f + _LOGC[1]
    p = p * f + _LOGC[0]
    return e.astype(jnp.float32) * LN2 + (p * (f * f) + f)


def _softplus(x):
    t = jnp.exp(-jnp.abs(x))
    p = jnp.float32(_LOGC[2])
    p = p * t + _LOGC[1]
    p = p * t + _LOGC[0]
    return jnp.maximum(x, 0.0) + (p * (t * t) + t)


def _body(x_hbm, tl_hbm, ns_hbm, out_hbm, x_v, tl_v, ns_v, out_v,
          xsem, tsem, nsem):
    cid = lax.axis_index("c")
    sid = lax.axis_index("s")
    wid = sid * 2 + cid

    def _addr(j):
        b = 2 * wid + j // 5
        h0 = (j % 5) * HROWS
        return b, h0

    def _issue(j, p):
        b, h0 = _addr(j)
        pltpu.async_copy(x_hbm.at[b, :, pl.ds(h0, HROWS), :], x_v.at[p], xsem)
        pltpu.async_copy(tl_hbm.at[b, pl.ds(h0, HROWS)], tl_v.at[p], tsem)
        pltpu.async_copy(ns_hbm.at[b, pl.ds(h0, HROWS)], ns_v.at[p], nsem)

    _issue(0, 0)

    def slab_body(j, _):
        p = j % 2
        b, h0 = _addr(j)
        pltpu.make_async_copy(
            x_hbm.at[b, :, pl.ds(h0, HROWS), :], x_v.at[p], xsem).wait()
        pltpu.make_async_copy(
            tl_hbm.at[b, pl.ds(h0, HROWS)], tl_v.at[p], tsem).wait()
        pltpu.make_async_copy(
            ns_hbm.at[b, pl.ds(h0, HROWS)], ns_v.at[p], nsem).wait()

        @pl.when(j < NSLAB - 1)
        def _prefetch():
            _issue(j + 1, 1 - p)

        def px_group(i, _):
            r = i // 5
            c0 = (i % 5) * 16
            s = pl.ds(c0, 16)
            n = ns_v[p, r, s]
            on_b = n > 0
            valid = [t < n for t in range(M)]

            tx = [tl_v[p, r, 2 * t, s] for t in range(M)]
            ty = [tl_v[p, r, 2 * t + 1, s] for t in range(M)]

            s_spl = jnp.zeros((16,), jnp.float32)
            s_l = jnp.zeros((16,), jnp.float32)
            s_nll = jnp.zeros((16,), jnp.float32)
            for k in range(M):
                l = x_v[p, 5 * k, r, s]
                mux = x_v[p, 5 * k + 1, r, s]
                muy = x_v[p, 5 * k + 2, r, s]
                sdxr = x_v[p, 5 * k + 3, r, s]
                sdyr = x_v[p, 5 * k + 4, r, s]

                ex = 1.0 / (1.0 + jnp.exp(-mux))
                ey = 1.0 / (1.0 + jnp.exp(-muy))
                sdx = _softplus(sdxr) + 1e-4
                sdy = _softplus(sdyr) + 1e-4

                dx = ex - tx[0]
                dy = ey - ty[0]
                best = jnp.where(valid[0], dx * dx + dy * dy, 1e9)
                bx = tx[0]
                by = ty[0]
                for t in range(1, M):
                    dx = ex - tx[t]
                    dy = ey - ty[t]
                    d2 = jnp.where(valid[t], dx * dx + dy * dy, 1e9)
                    mlt = d2 < best
                    best = jnp.where(mlt, d2, best)
                    bx = jnp.where(mlt, tx[t], bx)
                    by = jnp.where(mlt, ty[t], by)

                zx = (bx - ex) / sdx
                zy = (by - ey) / sdy
                s_spl = s_spl + _softplus(l)
                s_l = s_l + l
                s_nll = s_nll + (0.5 * (zx * zx + zy * zy)
                                 + _flog(sdx * sdy) + LOG2PI)

            out_v[r, s] = jnp.where(on_b, s_spl - s_l + s_nll, 0.1 * s_spl)
            return 0

        lax.fori_loop(0, SGROUPS, px_group, 0, unroll=False)
        pltpu.sync_copy(out_v, out_hbm.at[b, pl.ds(h0, HROWS)])
        return 0

    lax.fori_loop(0, NSLAB, slab_body, 0, unroll=False)


@jax.jit
def _sc_nll(x2d, tl2d, ns3d):
    mesh = plsc.VectorSubcoreMesh(core_axis_name="c", subcore_axis_name="s")
    f = functools.partial(
        pl.kernel,
        mesh=mesh,
        compiler_params=pltpu.CompilerParams(needs_layout_passes=False),
        out_type=jax.ShapeDtypeStruct((B, H, W), jnp.float32),
        scratch_types=[
            pltpu.VMEM((2, CH, HROWS, W), jnp.float32),
            pltpu.VMEM((2, HROWS, 2 * M, W), jnp.float32),
            pltpu.VMEM((2, HROWS, W), jnp.int32),
            pltpu.VMEM((HROWS, W), jnp.float32),
            pltpu.SemaphoreType.DMA,
            pltpu.SemaphoreType.DMA,
            pltpu.SemaphoreType.DMA,
        ],
    )(_body)
    return f(x2d, tl2d, ns3d)


def kernel(x_cat, true_locs, true_n_sources, topk):
    x4 = jnp.transpose(x_cat, (0, 3, 1, 2))
    tl4 = jnp.transpose(true_locs, (0, 1, 3, 4, 2)).reshape(B, H, 2 * M, W)
    ns3d = true_n_sources.astype(jnp.int32)
    return _sc_nll(x4, tl4, ns3d)
